# baseline (speedup 1.0000x reference)
import jax
import jax.numpy as jnp
from jax import lax
from jax.experimental import pallas as pl
from jax.experimental.pallas import tpu as pltpu
from jax.experimental.pallas import tpu_sc as plsc

SEQ = 512
DIAG = 2
NROWS = SEQ - DIAG
TRI = NROWS * (NROWS + 1) // 2
MATW = SEQ * SEQ
NMAT = 128
NC, NS = 2, 16
NW = NC * NS
MPW = NMAT // NW
R = 16
BLOCK_WORDS = R * SEQ


def _off(i: int) -> int:
    return NROWS * i - i * (i - 1) // 2


OFF = [_off(16 * b) for b in range(32)]
SB = [OFF[b + 1] - OFF[b] for b in range(31)]
FLUSH_LAST = 104
MAT_FLUSHED = OFF[31] + FLUSH_LAST
PATCH_CHUNKS = (260088, 260600, 261112)
OUT_WORDS = NMAT * TRI


def _body(in_hbm, out_hbm, in_buf, out_buf):
    c = lax.axis_index("c")
    s = lax.axis_index("s")
    w = s * NC + c
    par = w % 2
    m0 = w * MPW
    span = m0 * TRI - 4 * par
    c0 = 4 * par

    @pl.when(par == 1)
    def _head_patch():
        pbase = (m0 - 1) * MATW
        for n, chunk in enumerate(PATCH_CHUNKS):
            pltpu.sync_copy(in_hbm.at[pl.ds(pl.multiple_of(pbase + chunk, 8), 8)],
                            in_buf.at[pl.ds(8 * n, 8)])
        v0 = in_buf[pl.ds(0, 16)]
        v1 = in_buf[pl.ds(8, 16)]
        v2 = in_buf[pl.ds(16, 16)]
        lane = lax.iota(jnp.int32, 16)
        patch = jnp.where(lane == 0, v0[7],
                          jnp.where(lane == 1, v1[6],
                                    jnp.where(lane == 2, v1[7], v2[7])))
        out_buf[pl.ds(0, 16)] = patch

    def compact_rows(r0, cpos):
        def row_body(j, _):
            i = r0 + j
            length = NROWS - i
            nk = (length + 15) // 16
            src0 = SEQ * j + i + DIAG
            dst0 = cpos + (NROWS * i - i * (i - 1) // 2) - _off(r0)

            def k_body(k, __):
                out_buf[pl.ds(dst0 + 16 * k, 16)] = (
                    in_buf[pl.ds(src0 + 16 * k, 16)])
                return 0

            lax.fori_loop(0, nk, k_body, 0)
            return 0

        lax.fori_loop(0, R, row_body, 0)

    def mat_body(mat, _):
        mbase = (m0 + mat) * MATW
        cpos = c0 + mat
        obase = span + mat * MAT_FLUSHED
        for b in range(31):
            r0 = R * b
            pltpu.sync_copy(
                in_hbm.at[pl.ds(pl.multiple_of(mbase + SEQ * r0, 8),
                                BLOCK_WORDS)],
                in_buf.at[pl.ds(0, BLOCK_WORDS)])
            compact_rows(r0, cpos)
            pltpu.sync_copy(
                out_buf.at[pl.ds(0, SB[b])],
                out_hbm.at[pl.ds(pl.multiple_of(obase + OFF[b], 8), SB[b])])
            out_buf[pl.ds(0, 16)] = out_buf[pl.ds(SB[b], 16)]
        r0 = R * 31
        pltpu.sync_copy(
            in_hbm.at[pl.ds(pl.multiple_of(mbase + SEQ * r0, 8),
                            BLOCK_WORDS)],
            in_buf.at[pl.ds(0, BLOCK_WORDS)])
        compact_rows(r0, cpos)
        pltpu.sync_copy(
            out_buf.at[pl.ds(0, FLUSH_LAST)],
            out_hbm.at[pl.ds(pl.multiple_of(obase + OFF[31], 8), FLUSH_LAST)])
        out_buf[pl.ds(0, 16)] = out_buf[pl.ds(FLUSH_LAST, 16)]
        return 0

    lax.fori_loop(0, MPW, mat_body, 0)

    @pl.when(par == 1)
    def _tail_flush():
        pltpu.sync_copy(
            out_buf.at[pl.ds(0, 8)],
            out_hbm.at[pl.ds(pl.multiple_of(span + MPW * MAT_FLUSHED, 8), 8)])


@jax.jit
def _upper_tri(flat_in):
    mesh = plsc.VectorSubcoreMesh(core_axis_name="c", subcore_axis_name="s",
                                  num_cores=NC, num_subcores=NS)
    return pl.kernel(
        _body,
        out_type=jax.ShapeDtypeStruct((OUT_WORDS,), jnp.float32),
        mesh=mesh,
        scratch_types=[
            pltpu.VMEM((BLOCK_WORDS + 32,), jnp.float32),
            pltpu.VMEM((SB[0] + 40,), jnp.float32),
        ],
    )(flat_in)


def kernel(inputs):
    batch, chan, seq, _ = inputs.shape
    flat = inputs.reshape(batch * chan * seq * seq)
    out = _upper_tri(flat)
    return out.reshape(batch, chan, TRI)


# speedup vs baseline: 1.2882x; 1.2882x over previous
import jax
import jax.numpy as jnp
from jax import lax
from jax.experimental import pallas as pl
from jax.experimental.pallas import tpu as pltpu
from jax.experimental.pallas import tpu_sc as plsc

SEQ = 512
DIAG = 2
NROWS = SEQ - DIAG
TRI = NROWS * (NROWS + 1) // 2
NMAT = 128
NC, NS = 2, 16
NW = NC * NS
MPW = NMAT // NW
R = 16
NBLK = 32
IN_PITCH = SEQ + 32


def _off(i: int) -> int:
    return NROWS * i - i * (i - 1) // 2


OFF = [_off(R * b) for b in range(NBLK)]
FLUSH_LAST = 104
SBF = [OFF[b + 1] - OFF[b] for b in range(NBLK - 1)] + [FLUSH_LAST]
MAT_FLUSHED = OFF[NBLK - 1] + FLUSH_LAST
MATW = SEQ * SEQ
BLOCK_WORDS = R * SEQ
OUT_WORDS = NMAT * TRI
OUT_STAGE = SBF[0] + 40


def _body(in_hbm, out_hbm, in_a, in_b, out_a, out_b, si_a, si_b, so_a, so_b):
    c = lax.axis_index("c")
    s = lax.axis_index("s")
    w = s * NC + c
    par = w % 2
    m0 = w * MPW
    span = m0 * TRI - 4 * par
    c0 = 4 * par
    ibufs = (in_a, in_b)
    obufs = (out_a, out_b)
    isems = (si_a, si_b)
    osems = (so_a, so_b)

    @pl.when(par == 1)
    def _head_patch():
        pbase = (m0 - 1) * MATW
        for n, chunk in enumerate((260088, 260600, 261112)):
            pltpu.sync_copy(in_hbm.at[pl.ds(pl.multiple_of(pbase + chunk, 8),
                                            8)],
                            in_a.at[pl.ds(8 * n, 8)])
        v0 = in_a[pl.ds(0, 16)]
        v1 = in_a[pl.ds(8, 16)]
        v2 = in_a[pl.ds(16, 16)]
        lane = lax.iota(jnp.int32, 16)
        patch = jnp.where(lane == 0, v0[7],
                          jnp.where(lane == 1, v1[6],
                                    jnp.where(lane == 2, v1[7], v2[7])))
        out_b[pl.ds(FLUSH_LAST, 16)] = patch

    def in_dma(mbase, b, buf, sem):
        src = in_hbm.at[pl.ds(pl.multiple_of(mbase + SEQ * R * b, 8),
                              BLOCK_WORDS)]
        return pltpu.async_copy(src, buf.at[pl.ds(0, BLOCK_WORDS)], sem)

    def compact(b, cpos, ibuf, obuf):
        r0 = R * b

        def row_body(j, _):
            i = r0 + j
            length = NROWS - i
            nk = (length + 15) // 16
            dst0 = cpos + (NROWS * i - i * (i - 1) // 2) - _off(r0)
            src0 = SEQ * j + i + DIAG

            @plsc.parallel_loop(0, nk, 1, unroll=4)
            def _k(k):
                obuf[pl.ds(dst0 + 16 * k, 16)] = ibuf[pl.ds(src0 + 16 * k, 16)]

            return 0

        lax.fori_loop(0, R, row_body, 0)

    def mat_body(mat, _):
        mbase = (m0 + mat) * MATW
        cpos = c0 + mat
        obase = span + mat * MAT_FLUSHED
        h_in = [None] * NBLK
        h_out = [None] * NBLK
        h_in[0] = in_dma(mbase, 0, ibufs[0], isems[0])
        for b in range(NBLK):
            cur = b % 2
            if b + 1 < NBLK:
                h_in[b + 1] = in_dma(mbase, b + 1, ibufs[1 - cur],
                                     isems[1 - cur])
            h_in[b].wait()
            if b >= 2:
                h_out[b - 2].wait()
            prev_flush = FLUSH_LAST if b == 0 else SBF[b - 1]
            obufs[cur][pl.ds(0, 16)] = obufs[1 - cur][pl.ds(prev_flush, 16)]
            compact(b, cpos, ibufs[cur], obufs[cur])
            h_out[b] = pltpu.async_copy(
                obufs[cur].at[pl.ds(0, SBF[b])],
                out_hbm.at[pl.ds(pl.multiple_of(obase + OFF[b], 8), SBF[b])],
                osems[cur])
        h_out[NBLK - 2].wait()
        h_out[NBLK - 1].wait()
        return 0

    lax.fori_loop(0, MPW, mat_body, 0)

    @pl.when(par == 1)
    def _tail_flush():
        pltpu.sync_copy(
            obufs[1].at[pl.ds(FLUSH_LAST, 8)],
            out_hbm.at[pl.ds(pl.multiple_of(span + MPW * MAT_FLUSHED, 8), 8)])


@jax.jit
def _upper_tri(flat_in):
    mesh = plsc.VectorSubcoreMesh(core_axis_name="c", subcore_axis_name="s",
                                  num_cores=NC, num_subcores=NS)
    return pl.kernel(
        _body,
        out_type=jax.ShapeDtypeStruct((OUT_WORDS,), jnp.float32),
        mesh=mesh,
        scratch_types=[
            pltpu.VMEM((BLOCK_WORDS + 32,), jnp.float32),
            pltpu.VMEM((BLOCK_WORDS + 32,), jnp.float32),
            pltpu.VMEM((OUT_STAGE,), jnp.float32),
            pltpu.VMEM((OUT_STAGE,), jnp.float32),
            pltpu.SemaphoreType.DMA,
            pltpu.SemaphoreType.DMA,
            pltpu.SemaphoreType.DMA,
            pltpu.SemaphoreType.DMA,
        ],
    )(flat_in)


def kernel(inputs):
    batch, chan, seq, _ = inputs.shape
    flat = inputs.reshape(batch * chan * seq * seq)
    out = _upper_tri(flat)
    return out.reshape(batch, chan, TRI)


# speedup vs baseline: 1.3333x; 1.0350x over previous
import jax
import jax.numpy as jnp
from jax import lax
from jax.experimental import pallas as pl
from jax.experimental.pallas import tpu as pltpu
from jax.experimental.pallas import tpu_sc as plsc

SEQ = 512
DIAG = 2
NROWS = SEQ - DIAG
TRI = NROWS * (NROWS + 1) // 2
NMAT = 128
NC, NS = 2, 16
NW = NC * NS
MPW = NMAT // NW
R = 32
NBLK = 16
NRING = 3


def _off(i: int) -> int:
    return NROWS * i - i * (i - 1) // 2


OFF = [_off(R * b) for b in range(NBLK)]
FLUSH_LAST = 464
SBF = [OFF[b + 1] - OFF[b] for b in range(NBLK - 1)] + [FLUSH_LAST]
MAT_FLUSHED = OFF[NBLK - 1] + FLUSH_LAST
MATW = SEQ * SEQ
BLOCK_WORDS = R * SEQ
OUT_WORDS = NMAT * TRI
OUT_STAGE = SBF[0] + 40


def _body(in_hbm, out_hbm, in_a, in_b, in_c, out_a, out_b,
          si_a, si_b, si_c, so_a, so_b):
    c = lax.axis_index("c")
    s = lax.axis_index("s")
    w = s * NC + c
    par = w % 2
    m0 = w * MPW
    span = m0 * TRI - 4 * par
    c0 = 4 * par
    ibufs = (in_a, in_b, in_c)
    obufs = (out_a, out_b)
    isems = (si_a, si_b, si_c)
    osems = (so_a, so_b)

    @pl.when(par == 1)
    def _head_patch():
        pbase = (m0 - 1) * MATW
        for n, chunk in enumerate((260088, 260600, 261112)):
            pltpu.sync_copy(in_hbm.at[pl.ds(pl.multiple_of(pbase + chunk, 8),
                                            8)],
                            in_a.at[pl.ds(8 * n, 8)])
        v0 = in_a[pl.ds(0, 16)]
        v1 = in_a[pl.ds(8, 16)]
        v2 = in_a[pl.ds(16, 16)]
        lane = lax.iota(jnp.int32, 16)
        patch = jnp.where(lane == 0, v0[7],
                          jnp.where(lane == 1, v1[6],
                                    jnp.where(lane == 2, v1[7], v2[7])))
        out_b[pl.ds(FLUSH_LAST, 16)] = patch

    def in_dma(mbase, b, buf, sem):
        src = in_hbm.at[pl.ds(pl.multiple_of(mbase + SEQ * R * b, 8),
                              BLOCK_WORDS)]
        return pltpu.async_copy(src, buf.at[pl.ds(0, BLOCK_WORDS)], sem)

    def compact(b, cpos, ibuf, obuf):
        r0 = R * b

        def row_body(j, _):
            i = r0 + j
            length = NROWS - i
            nk = (length + 15) // 16
            dst0 = cpos + (NROWS * i - i * (i - 1) // 2) - _off(r0)
            src0 = SEQ * j + i + DIAG

            @plsc.parallel_loop(0, nk, 1, unroll=4)
            def _k(k):
                obuf[pl.ds(dst0 + 16 * k, 16)] = ibuf[pl.ds(src0 + 16 * k, 16)]

            return 0

        lax.fori_loop(0, R, row_body, 0)

    def mat_body(mat, _):
        mbase = (m0 + mat) * MATW
        cpos = c0 + mat
        obase = span + mat * MAT_FLUSHED
        h_in = [None] * NBLK
        h_out = [None] * NBLK
        for p in range(NRING - 1):
            h_in[p] = in_dma(mbase, p, ibufs[p], isems[p])
        for b in range(NBLK):
            cur = b % 2
            if b + NRING - 1 < NBLK:
                ring = (b + NRING - 1) % NRING
                h_in[b + NRING - 1] = in_dma(mbase, b + NRING - 1,
                                             ibufs[ring], isems[ring])
            h_in[b].wait()
            if b >= 2:
                h_out[b - 2].wait()
            prev_flush = FLUSH_LAST if b == 0 else SBF[b - 1]
            obufs[cur][pl.ds(0, 16)] = obufs[1 - cur][pl.ds(prev_flush, 16)]
            compact(b, cpos, ibufs[b % NRING], obufs[cur])
            h_out[b] = pltpu.async_copy(
                obufs[cur].at[pl.ds(0, SBF[b])],
                out_hbm.at[pl.ds(pl.multiple_of(obase + OFF[b], 8), SBF[b])],
                osems[cur])
        h_out[NBLK - 2].wait()
        h_out[NBLK - 1].wait()
        return 0

    lax.fori_loop(0, MPW, mat_body, 0)

    @pl.when(par == 1)
    def _tail_flush():
        pltpu.sync_copy(
            obufs[1].at[pl.ds(FLUSH_LAST, 8)],
            out_hbm.at[pl.ds(pl.multiple_of(span + MPW * MAT_FLUSHED, 8), 8)])


@jax.jit
def _upper_tri(flat_in):
    mesh = plsc.VectorSubcoreMesh(core_axis_name="c", subcore_axis_name="s",
                                  num_cores=NC, num_subcores=NS)
    return pl.kernel(
        _body,
        out_type=jax.ShapeDtypeStruct((OUT_WORDS,), jnp.float32),
        mesh=mesh,
        scratch_types=[
            pltpu.VMEM((BLOCK_WORDS + 32,), jnp.float32),
            pltpu.VMEM((BLOCK_WORDS + 32,), jnp.float32),
            pltpu.VMEM((BLOCK_WORDS + 32,), jnp.float32),
            pltpu.VMEM((OUT_STAGE,), jnp.float32),
            pltpu.VMEM((OUT_STAGE,), jnp.float32),
            pltpu.SemaphoreType.DMA,
            pltpu.SemaphoreType.DMA,
            pltpu.SemaphoreType.DMA,
            pltpu.SemaphoreType.DMA,
            pltpu.SemaphoreType.DMA,
        ],
    )(flat_in)


def kernel(inputs):
    batch, chan, seq, _ = inputs.shape
    flat = inputs.reshape(batch * chan * seq * seq)
    out = _upper_tri(flat)
    return out.reshape(batch, chan, TRI)
